# baseline (speedup 1.0000x reference)
import jax
import jax.numpy as jnp
from jax.experimental import pallas as pl
from jax.experimental.pallas import tpu as pltpu

N = 4096
C = 512
HD = 512
K_NEIGH = 16
QBLK = 256
NBLK = N // QBLK
SCALE = HD ** (-0.5)


def _body(h_ref, dt_ref, dp_ref, wq_ref, wk_ref, wv_ref, wo_ref, bo_ref,
          o_ref, k_scr, v_scr, t_scr):
    i = pl.program_id(0)

    @pl.when(i == 0)
    def _():
        h_all = h_ref[...].astype(jnp.bfloat16)
        k_scr[...] = jax.lax.dot(h_all, wk_ref[...].astype(jnp.bfloat16),
                                 preferred_element_type=jnp.float32
                                 ).astype(jnp.bfloat16)
        v_scr[...] = jax.lax.dot(h_all, wv_ref[...].astype(jnp.bfloat16),
                                 preferred_element_type=jnp.float32
                                 ).astype(jnp.bfloat16)

    t_prev = t_scr[(i + 1) % 2]

    d = dt_ref[...]
    t = jnp.full((QBLK, 1), -jnp.inf, dtype=jnp.float32)
    for _ in range(K_NEIGH):
        t = jnp.min(jnp.where(d > t, d, jnp.inf), axis=1, keepdims=True)
    t_scr[i % 2] = t

    j = jnp.maximum(i - 1, 0)
    hb = h_ref[pl.ds(j * QBLK, QBLK), :]
    q = jax.lax.dot(hb.astype(jnp.bfloat16),
                    wq_ref[...].astype(jnp.bfloat16),
                    preferred_element_type=jnp.float32
                    ).astype(jnp.bfloat16)
    s = jax.lax.dot_general(q, k_scr[...], (((1,), (1,)), ((), ())),
                            preferred_element_type=jnp.float32) * SCALE
    mask = dp_ref[...] <= t_prev
    logits = jnp.where(mask, s, -jnp.inf)
    mx = jnp.max(logits, axis=1, keepdims=True)
    p = jnp.exp(logits - mx)
    attn = p / jnp.sum(p, axis=1, keepdims=True)
    he = jax.lax.dot_general(attn.astype(jnp.bfloat16), v_scr[...],
                             (((1,), (0,)), ((), ())),
                             preferred_element_type=jnp.float32
                             ).astype(jnp.bfloat16)
    o_ref[...] = (jax.lax.dot(he, wo_ref[...].astype(jnp.bfloat16),
                              preferred_element_type=jnp.float32)
                  + bo_ref[...] + hb)


@jax.jit
def kernel(H, distance_matrix, Wq, Wk, Wv, Wo, bo):
    last = NBLK - 1
    out = pl.pallas_call(
        _body,
        grid=(NBLK + 1,),
        in_specs=[
            pl.BlockSpec((N, C), lambda i: (0, 0)),
            pl.BlockSpec((QBLK, N),
                         lambda i: (jnp.minimum(i, last), 0)),
            pl.BlockSpec((QBLK, N),
                         lambda i: (jnp.maximum(i - 1, 0), 0)),
            pl.BlockSpec((C, HD), lambda i: (0, 0)),
            pl.BlockSpec((C, HD), lambda i: (0, 0)),
            pl.BlockSpec((C, C), lambda i: (0, 0)),
            pl.BlockSpec((C, C), lambda i: (0, 0)),
            pl.BlockSpec((1, C), lambda i: (0, 0)),
        ],
        out_specs=pl.BlockSpec((QBLK, C), lambda i: (jnp.maximum(i - 1, 0), 0)),
        out_shape=jax.ShapeDtypeStruct((N, C), jnp.float32),
        scratch_shapes=[
            pltpu.VMEM((N, HD), jnp.bfloat16),
            pltpu.VMEM((N, C), jnp.bfloat16),
            pltpu.VMEM((2, QBLK, 1), jnp.float32),
        ],
    )(H, distance_matrix, distance_matrix, Wq, Wk, Wv, Wo, bo.reshape(1, C))
    return out


# speedup vs baseline: 1.9713x; 1.9713x over previous
import jax
import jax.numpy as jnp
from jax.experimental import pallas as pl
from jax.experimental.pallas import tpu as pltpu

N = 4096
C = 512
HD = 512
K_NEIGH = 16
QBLK = 256
NBLK = N // QBLK
NPOS = 128
NCHUNK = N // NPOS
SCALE = HD ** (-0.5)


def _body(h_ref, d_ref, wq_ref, wk_ref, wv_ref, wo_ref, bo_ref, o_ref,
          k_scr, v_scr):
    i = pl.program_id(0)

    @pl.when(i == 0)
    def _():
        h_all = h_ref[...].astype(jnp.bfloat16)
        k_scr[...] = jax.lax.dot(h_all, wk_ref[...].astype(jnp.bfloat16),
                                 preferred_element_type=jnp.float32
                                 ).astype(jnp.bfloat16)
        v_scr[...] = jax.lax.dot(h_all, wv_ref[...].astype(jnp.bfloat16),
                                 preferred_element_type=jnp.float32
                                 ).astype(jnp.bfloat16)

    hb = h_ref[pl.ds(i * QBLK, QBLK), :]
    q = jax.lax.dot(hb.astype(jnp.bfloat16),
                    wq_ref[...].astype(jnp.bfloat16),
                    preferred_element_type=jnp.float32).astype(jnp.bfloat16)

    d = d_ref[...]

    inf = jnp.full((QBLK, NPOS), jnp.inf, dtype=jnp.float32)
    m1, m2, m3, m4 = inf, inf, inf, inf
    for c in range(NCHUNK):
        v = d[:, c * NPOS:(c + 1) * NPOS]
        l1 = jnp.minimum(m1, v)
        c1 = jnp.maximum(m1, v)
        l2 = jnp.minimum(m2, c1)
        c2 = jnp.maximum(m2, c1)
        l3 = jnp.minimum(m3, c2)
        c3 = jnp.maximum(m3, c2)
        l4 = jnp.minimum(m4, c3)
        m1, m2, m3, m4 = l1, l2, l3, l4
    cand = jnp.concatenate([m1, m2, m3, m4], axis=1)

    t = jnp.full((QBLK, 1), -jnp.inf, dtype=jnp.float32)
    for _ in range(K_NEIGH):
        t = jnp.min(jnp.where(cand > t, cand, jnp.inf), axis=1, keepdims=True)
    mask = d <= t

    s = jax.lax.dot_general(q, k_scr[...], (((1,), (1,)), ((), ())),
                            preferred_element_type=jnp.float32) * SCALE
    logits = jnp.where(mask, s, -jnp.inf)
    mx = jnp.max(logits, axis=1, keepdims=True)
    p = jnp.exp(logits - mx)
    attn = p / jnp.sum(p, axis=1, keepdims=True)

    he = jax.lax.dot_general(attn.astype(jnp.bfloat16), v_scr[...],
                             (((1,), (0,)), ((), ())),
                             preferred_element_type=jnp.float32
                             ).astype(jnp.bfloat16)
    o_ref[...] = (jax.lax.dot(he, wo_ref[...].astype(jnp.bfloat16),
                              preferred_element_type=jnp.float32)
                  + bo_ref[...] + hb)


@jax.jit
def kernel(H, distance_matrix, Wq, Wk, Wv, Wo, bo):
    out = pl.pallas_call(
        _body,
        grid=(NBLK,),
        in_specs=[
            pl.BlockSpec((N, C), lambda i: (0, 0)),
            pl.BlockSpec((QBLK, N), lambda i: (i, 0)),
            pl.BlockSpec((C, HD), lambda i: (0, 0)),
            pl.BlockSpec((C, HD), lambda i: (0, 0)),
            pl.BlockSpec((C, C), lambda i: (0, 0)),
            pl.BlockSpec((C, C), lambda i: (0, 0)),
            pl.BlockSpec((1, C), lambda i: (0, 0)),
        ],
        out_specs=pl.BlockSpec((QBLK, C), lambda i: (i, 0)),
        out_shape=jax.ShapeDtypeStruct((N, C), jnp.float32),
        scratch_shapes=[
            pltpu.VMEM((N, HD), jnp.bfloat16),
            pltpu.VMEM((N, C), jnp.bfloat16),
        ],
    )(H, distance_matrix, Wq, Wk, Wv, Wo, bo.reshape(1, C))
    return out
